# baseline (speedup 1.0000x reference)
import functools

import jax
import jax.numpy as jnp
from jax import lax
from jax.experimental import pallas as pl
from jax.experimental.pallas import tpu as pltpu
from jax.experimental.pallas import tpu_sc as plsc

B, R, N, K = 8, 32, 128, 64
ROWS = B * R
NS, L = 8, 16
NW = NS
RPW = ROWS // NW
WPB = R // RPW
EPW = RPW * K
VECS = EPW // L
VPR = K // L

_mesh = plsc.VectorSubcoreMesh(
    core_axis_name="c", subcore_axis_name="s", num_cores=1, num_subcores=8
)


@functools.partial(
    pl.kernel,
    mesh=_mesh,
    out_type=jax.ShapeDtypeStruct((B, R, K), jnp.float32),
    scratch_types=[
        pltpu.VMEM((RPW * N,), jnp.float32),
        pltpu.VMEM((EPW,), jnp.int32),
        pltpu.VMEM((RPW, K), jnp.float32),
        pltpu.SemaphoreType.DMA,
        pltpu.SemaphoreType.DMA,
        pltpu.SemaphoreType.DMA,
    ],
    compiler_params=pltpu.CompilerParams(needs_layout_passes=False),
)
def _gather_sc(x_hbm, idx_hbm, out_hbm, x_v, idx_v, o_v, sem_x, sem_i, sem_o):
    wid = lax.axis_index("s")
    cp_x = pltpu.async_copy(x_hbm.at[pl.ds(wid * RPW * N, RPW * N)], x_v, sem_x)
    cp_i = pltpu.async_copy(idx_hbm.at[pl.ds(wid * EPW, EPW)], idx_v, sem_i)
    cp_i.wait()
    cp_x.wait()
    b, r0 = wid // WPB, (wid % WPB) * RPW
    half = RPW // 2
    cps = []
    for h in range(2):
        for i in range(h * VECS // 2, (h + 1) * VECS // 2):
            r = i // VPR
            idx = idx_v[pl.ds(i * L, L)] + r * N
            o_v[r, pl.ds((i % VPR) * L, L)] = plsc.load_gather(x_v, [idx])
        cps.append(
            pltpu.async_copy(
                o_v.at[pl.ds(h * half, half), :],
                out_hbm.at[b, pl.ds(r0 + h * half, half), :],
                sem_o,
            )
        )
    for cp in cps:
        cp.wait()


def kernel(x, index, dim):
    del dim
    xf = x.reshape(-1).astype(jnp.float32)
    idxf = index.astype(jnp.int32).reshape(-1)
    out = _gather_sc(xf, idxf)
    return out.astype(x.dtype)


# speedup vs baseline: 1.0485x; 1.0485x over previous
import functools

import jax
import jax.numpy as jnp
from jax import lax
from jax.experimental import pallas as pl
from jax.experimental.pallas import tpu as pltpu
from jax.experimental.pallas import tpu_sc as plsc

B, R, N, K = 8, 32, 128, 64
ROWS = B * R
NS, L = 16, 16
NW = NS
RPW = ROWS // NW
WPB = R // RPW
EPW = RPW * K
VECS = EPW // L
VPR = K // L

_mesh = plsc.VectorSubcoreMesh(
    core_axis_name="c", subcore_axis_name="s", num_cores=1
)


@functools.partial(
    pl.kernel,
    mesh=_mesh,
    out_type=jax.ShapeDtypeStruct((B, R, K), jnp.float32),
    scratch_types=[
        pltpu.VMEM((RPW * N,), jnp.float32),
        pltpu.VMEM((EPW,), jnp.int32),
        pltpu.VMEM((RPW, K), jnp.float32),
        pltpu.SemaphoreType.DMA,
        pltpu.SemaphoreType.DMA,
        pltpu.SemaphoreType.DMA,
    ],
    compiler_params=pltpu.CompilerParams(needs_layout_passes=False),
)
def _gather_sc(x_hbm, idx_hbm, out_hbm, x_v, idx_v, o_v, sem_x, sem_i, sem_o):
    wid = lax.axis_index("s")
    cp_x = pltpu.async_copy(x_hbm.at[pl.ds(wid * RPW * N, RPW * N)], x_v, sem_x)
    cp_i = pltpu.async_copy(idx_hbm.at[pl.ds(wid * EPW, EPW)], idx_v, sem_i)
    cp_i.wait()
    cp_x.wait()
    b, r0 = wid // WPB, (wid % WPB) * RPW
    half = RPW // 2
    cps = []
    for h in range(2):
        for i in range(h * VECS // 2, (h + 1) * VECS // 2):
            r = i // VPR
            idx = idx_v[pl.ds(i * L, L)] + r * N
            o_v[r, pl.ds((i % VPR) * L, L)] = plsc.load_gather(x_v, [idx])
        cps.append(
            pltpu.async_copy(
                o_v.at[pl.ds(h * half, half), :],
                out_hbm.at[b, pl.ds(r0 + h * half, half), :],
                sem_o,
            )
        )
    for cp in cps:
        cp.wait()


def kernel(x, index, dim):
    del dim
    xf = x.reshape(-1).astype(jnp.float32)
    idxf = index.astype(jnp.int32).reshape(-1)
    out = _gather_sc(xf, idxf)
    return out.astype(x.dtype)
